# Initial kernel scaffold; baseline (speedup 1.0000x reference)
import jax
import jax.numpy as jnp
from jax.experimental import pallas as pl


def kernel(x, edge_index, W0, b0, W1, b1):
    raise NotImplementedError("write your pallas kernel here")


# speedup vs baseline: 11.5639x; 11.5639x over previous
import functools

import jax
import jax.numpy as jnp
from jax import lax
from jax.experimental import pallas as pl
from jax.experimental.pallas import tpu as pltpu
from jax.experimental.pallas import tpu_sc as plsc

N = 10000
E = 320000
D_IN = 128
HID = 128
N_CLS = 64

NPAD = 10240
NW = 32
NB = 80
B = 128
EPAD = NW * NB * B
RPT = NPAD // 16
DUMMY = N + 16

MT = 1000


def _mm_body(x_ref, w_ref, o_ref):
    o_ref[...] = jnp.dot(x_ref[...], w_ref[...],
                         preferred_element_type=jnp.float32)


def _mm(x, w):
    m, k = x.shape
    n = w.shape[1]
    return pl.pallas_call(
        _mm_body,
        grid=(m // MT,),
        in_specs=[pl.BlockSpec((MT, k), lambda i: (i, 0)),
                  pl.BlockSpec((k, n), lambda i: (0, 0))],
        out_specs=pl.BlockSpec((MT, n), lambda i: (i, 0)),
        out_shape=jax.ShapeDtypeStruct((m, n), jnp.float32),
    )(x, w)


def _scale0_body(deg_ref, xw_ref, y_ref, dinv_ref):
    d = jnp.sum(deg_ref[...], axis=0) + 1.0
    dinv = lax.rsqrt(d)
    dinv_ref[...] = dinv
    y_ref[...] = xw_ref[...] * dinv


def _scale0(deg, xw):
    return pl.pallas_call(
        _scale0_body,
        grid=(N // MT,),
        in_specs=[pl.BlockSpec((2, MT, 1), lambda i: (0, i, 0)),
                  pl.BlockSpec((MT, HID), lambda i: (i, 0))],
        out_specs=[pl.BlockSpec((MT, HID), lambda i: (i, 0)),
                   pl.BlockSpec((MT, 1), lambda i: (i, 0))],
        out_shape=[jax.ShapeDtypeStruct((N, HID), jnp.float32),
                   jax.ShapeDtypeStruct((N, 1), jnp.float32)],
    )(deg, xw)


def _mid_body(acc_ref, y0_ref, dinv_ref, b0_ref, w1_ref, y1_ref):
    dinv = dinv_ref[...]
    h = acc_ref[0] + acc_ref[1] + y0_ref[...]
    h = jnp.maximum(dinv * h + b0_ref[...], 0.0)
    y1_ref[...] = jnp.dot(h, w1_ref[...],
                          preferred_element_type=jnp.float32) * dinv


def _mid(acc, y0, dinv, b0, w1):
    return pl.pallas_call(
        _mid_body,
        grid=(N // MT,),
        in_specs=[pl.BlockSpec((2, MT, HID), lambda i: (0, i, 0)),
                  pl.BlockSpec((MT, HID), lambda i: (i, 0)),
                  pl.BlockSpec((MT, 1), lambda i: (i, 0)),
                  pl.BlockSpec((1, HID), lambda i: (0, 0)),
                  pl.BlockSpec((HID, N_CLS), lambda i: (0, 0))],
        out_specs=pl.BlockSpec((MT, N_CLS), lambda i: (i, 0)),
        out_shape=jax.ShapeDtypeStruct((N, N_CLS), jnp.float32),
    )(acc, y0, dinv, b0, w1)


def _fin_body(acc_ref, y1_ref, dinv_ref, b1_ref, o_ref):
    o_ref[...] = (dinv_ref[...] * (acc_ref[0] + acc_ref[1] + y1_ref[...])
                  + b1_ref[...])


def _fin(acc, y1, dinv, b1):
    return pl.pallas_call(
        _fin_body,
        grid=(N // MT,),
        in_specs=[pl.BlockSpec((2, MT, N_CLS), lambda i: (0, i, 0)),
                  pl.BlockSpec((MT, N_CLS), lambda i: (i, 0)),
                  pl.BlockSpec((MT, 1), lambda i: (i, 0)),
                  pl.BlockSpec((1, N_CLS), lambda i: (0, 0))],
        out_specs=pl.BlockSpec((MT, N_CLS), lambda i: (i, 0)),
        out_shape=jax.ShapeDtypeStruct((N, N_CLS), jnp.float32),
    )(acc, y1, dinv, b1)


_MESH = plsc.VectorSubcoreMesh(core_axis_name="c", subcore_axis_name="s")


@functools.partial(
    pl.kernel, mesh=_MESH,
    out_type=jax.ShapeDtypeStruct((2, NPAD), jnp.float32),
    scratch_types=[pltpu.VMEM((NB, B), jnp.int32),
                   pltpu.VMEM((B,), jnp.float32),
                   pltpu.VMEM_SHARED((NPAD,), jnp.float32)])
def _deg_kernel(dst_hbm, zeros_hbm, out_hbm, dst_v, ones_v, acc_sp):
    c = lax.axis_index("c")
    s = lax.axis_index("s")
    w = c * 16 + s
    pltpu.sync_copy(zeros_hbm, acc_sp.at[pl.ds(s * RPT, RPT)])
    pltpu.sync_copy(dst_hbm.at[w], dst_v)
    for i in range(B // 16):
        ones_v[pl.ds(i * 16, 16)] = jnp.ones((16,), jnp.float32)
    plsc.subcore_barrier()

    def body(j, carry):
        pltpu.sync_copy(ones_v, acc_sp.at[dst_v.at[j]], add=True)
        return carry

    lax.fori_loop(0, NB, body, 0)
    plsc.subcore_barrier()
    pltpu.sync_copy(acc_sp.at[pl.ds(s * RPT, RPT)],
                    out_hbm.at[c, pl.ds(s * RPT, RPT)])


def _make_agg(F):
    @functools.partial(
        pl.kernel, mesh=_MESH,
        compiler_params=pltpu.CompilerParams(use_tc_tiling_on_sc=False),
        out_type=jax.ShapeDtypeStruct((2, NPAD, F), jnp.float32),
        scratch_types=[pltpu.VMEM((NB, B), jnp.int32),
                       pltpu.VMEM((NB, B), jnp.int32),
                       pltpu.VMEM((B, F), jnp.float32),
                       pltpu.VMEM_SHARED((NPAD, F), jnp.float32),
                       pltpu.SemaphoreType.DMA])
    def agg(y_hbm, src_hbm, dst_hbm, zeros_hbm, out_hbm,
            src_v, dst_v, rows_v, acc_sp, sem):
        c = lax.axis_index("c")
        s = lax.axis_index("s")
        w = c * 16 + s
        pltpu.sync_copy(zeros_hbm, acc_sp.at[pl.ds(s * RPT, RPT)])
        pltpu.sync_copy(src_hbm.at[w], src_v)
        pltpu.sync_copy(dst_hbm.at[w], dst_v)
        plsc.subcore_barrier()

        def body(j, carry):
            pltpu.async_copy(y_hbm.at[src_v.at[j]], rows_v, sem).wait()
            pltpu.sync_copy(rows_v, acc_sp.at[dst_v.at[j]], add=True)
            return carry

        lax.fori_loop(0, NB, body, 0)
        plsc.subcore_barrier()
        pltpu.sync_copy(acc_sp.at[pl.ds(s * RPT, RPT)],
                        out_hbm.at[c, pl.ds(s * RPT, RPT)])

    return agg


_agg_hid = _make_agg(HID)
_agg_cls = _make_agg(N_CLS)


def kernel(x, edge_index, W0, b0, W1, b1):
    src = edge_index[0]
    dst = edge_index[1]
    pad = EPAD - E
    srcp = jnp.concatenate(
        [src, jnp.zeros((pad,), jnp.int32)]).reshape(NW, NB, B)
    dstp = jnp.concatenate(
        [dst, jnp.full((pad,), DUMMY, jnp.int32)]).reshape(NW, NB, B)

    z_deg = jnp.zeros((RPT,), jnp.float32)
    deg = _deg_kernel(dstp, z_deg)
    xw0 = _mm(x, W0)
    y0, dinv = _scale0(deg[:, :N].reshape(2, N, 1), xw0)

    z_hid = jnp.zeros((RPT, HID), jnp.float32)
    z_cls = jnp.zeros((RPT, N_CLS), jnp.float32)

    acc0 = _agg_hid(y0, srcp, dstp, z_hid)
    y1 = _mid(acc0[:, :N, :], y0, dinv,
              b0.reshape(1, HID), W1)
    acc1 = _agg_cls(y1, srcp, dstp, z_cls)
    out = _fin(acc1[:, :N, :], y1, dinv, b1.reshape(1, N_CLS))
    return out


# speedup vs baseline: 11.9050x; 1.0295x over previous
import functools

import jax
import jax.numpy as jnp
from jax import lax
from jax.experimental import pallas as pl
from jax.experimental.pallas import tpu as pltpu
from jax.experimental.pallas import tpu_sc as plsc

N = 10000
E = 320000
D_IN = 128
HID = 128
N_CLS = 64

NPAD = 10240
NW = 32
NB = 80
B = 128
EPAD = NW * NB * B
RPT = NPAD // 16
DUMMY = N + 16

MT = 1000


def _mm_body(x_ref, w_ref, o_ref):
    o_ref[...] = jnp.dot(x_ref[...], w_ref[...],
                         preferred_element_type=jnp.float32)


def _mm(x, w):
    m, k = x.shape
    n = w.shape[1]
    return pl.pallas_call(
        _mm_body,
        grid=(m // MT,),
        in_specs=[pl.BlockSpec((MT, k), lambda i: (i, 0)),
                  pl.BlockSpec((k, n), lambda i: (0, 0))],
        out_specs=pl.BlockSpec((MT, n), lambda i: (i, 0)),
        out_shape=jax.ShapeDtypeStruct((m, n), jnp.float32),
    )(x, w)


def _scale0_body(deg_ref, xw_ref, y_ref, dinv_ref):
    d = jnp.sum(deg_ref[...], axis=0) + 1.0
    dinv = lax.rsqrt(d)
    dinv_ref[...] = dinv
    y_ref[...] = xw_ref[...] * dinv


def _scale0(deg, xw):
    return pl.pallas_call(
        _scale0_body,
        grid=(N // MT,),
        in_specs=[pl.BlockSpec((2, MT, 1), lambda i: (0, i, 0)),
                  pl.BlockSpec((MT, HID), lambda i: (i, 0))],
        out_specs=[pl.BlockSpec((MT, HID), lambda i: (i, 0)),
                   pl.BlockSpec((MT, 1), lambda i: (i, 0))],
        out_shape=[jax.ShapeDtypeStruct((N, HID), jnp.float32),
                   jax.ShapeDtypeStruct((N, 1), jnp.float32)],
    )(deg, xw)


def _mid_body(acc_ref, y0_ref, dinv_ref, b0_ref, w1_ref, y1_ref):
    dinv = dinv_ref[...]
    h = acc_ref[0] + acc_ref[1] + y0_ref[...]
    h = jnp.maximum(dinv * h + b0_ref[...], 0.0)
    y1_ref[...] = jnp.dot(h, w1_ref[...],
                          preferred_element_type=jnp.float32) * dinv


def _mid(acc, y0, dinv, b0, w1):
    return pl.pallas_call(
        _mid_body,
        grid=(N // MT,),
        in_specs=[pl.BlockSpec((2, MT, HID), lambda i: (0, i, 0)),
                  pl.BlockSpec((MT, HID), lambda i: (i, 0)),
                  pl.BlockSpec((MT, 1), lambda i: (i, 0)),
                  pl.BlockSpec((1, HID), lambda i: (0, 0)),
                  pl.BlockSpec((HID, N_CLS), lambda i: (0, 0))],
        out_specs=pl.BlockSpec((MT, N_CLS), lambda i: (i, 0)),
        out_shape=jax.ShapeDtypeStruct((N, N_CLS), jnp.float32),
    )(acc, y0, dinv, b0, w1)


def _fin_body(acc_ref, y1_ref, dinv_ref, b1_ref, o_ref):
    o_ref[...] = (dinv_ref[...] * (acc_ref[0] + acc_ref[1] + y1_ref[...])
                  + b1_ref[...])


def _fin(acc, y1, dinv, b1):
    return pl.pallas_call(
        _fin_body,
        grid=(N // MT,),
        in_specs=[pl.BlockSpec((2, MT, N_CLS), lambda i: (0, i, 0)),
                  pl.BlockSpec((MT, N_CLS), lambda i: (i, 0)),
                  pl.BlockSpec((MT, 1), lambda i: (i, 0)),
                  pl.BlockSpec((1, N_CLS), lambda i: (0, 0))],
        out_specs=pl.BlockSpec((MT, N_CLS), lambda i: (i, 0)),
        out_shape=jax.ShapeDtypeStruct((N, N_CLS), jnp.float32),
    )(acc, y1, dinv, b1)


_MESH = plsc.VectorSubcoreMesh(core_axis_name="c", subcore_axis_name="s")


@functools.partial(
    pl.kernel, mesh=_MESH,
    out_type=jax.ShapeDtypeStruct((2, NPAD), jnp.float32),
    scratch_types=[pltpu.VMEM((NB, B), jnp.int32),
                   pltpu.VMEM((B,), jnp.float32),
                   pltpu.VMEM_SHARED((NPAD,), jnp.float32)])
def _deg_kernel(dst_hbm, zeros_hbm, out_hbm, dst_v, ones_v, acc_sp):
    c = lax.axis_index("c")
    s = lax.axis_index("s")
    w = c * 16 + s
    pltpu.sync_copy(zeros_hbm, acc_sp.at[pl.ds(s * RPT, RPT)])
    pltpu.sync_copy(dst_hbm.at[w], dst_v)
    for i in range(B // 16):
        ones_v[pl.ds(i * 16, 16)] = jnp.ones((16,), jnp.float32)
    plsc.subcore_barrier()

    def body(j, carry):
        pltpu.sync_copy(ones_v, acc_sp.at[dst_v.at[j]], add=True)
        return carry

    lax.fori_loop(0, NB, body, 0)
    plsc.subcore_barrier()
    pltpu.sync_copy(acc_sp.at[pl.ds(s * RPT, RPT)],
                    out_hbm.at[c, pl.ds(s * RPT, RPT)])


def _make_agg(F):
    @functools.partial(
        pl.kernel, mesh=_MESH,
        compiler_params=pltpu.CompilerParams(use_tc_tiling_on_sc=False),
        out_type=jax.ShapeDtypeStruct((2, NPAD, F), jnp.float32),
        scratch_types=[pltpu.VMEM((NB // 2, B), jnp.int32),
                       pltpu.VMEM((NB // 2, B), jnp.int32),
                       pltpu.VMEM((B, F), jnp.float32),
                       pltpu.VMEM((B, F), jnp.float32),
                       pltpu.VMEM_SHARED((NPAD, F), jnp.float32),
                       pltpu.SemaphoreType.DMA,
                       pltpu.SemaphoreType.DMA])
    def agg(y_hbm, src_hbm, dst_hbm, zeros_hbm, out_hbm,
            src_v, dst_v, rows0_v, rows1_v, acc_sp, sem0, sem1):
        c = lax.axis_index("c")
        s = lax.axis_index("s")
        w = c * 16 + s
        nbh = NB // 2
        pltpu.sync_copy(zeros_hbm, acc_sp.at[pl.ds(s * RPT, RPT)])
        plsc.subcore_barrier()

        for h in range(2):
            pltpu.sync_copy(src_hbm.at[w, pl.ds(h * nbh, nbh)], src_v)
            pltpu.sync_copy(dst_hbm.at[w, pl.ds(h * nbh, nbh)], dst_v)
            pltpu.async_copy(y_hbm.at[src_v.at[0]], rows0_v, sem0)

            def body(i, carry):
                j = 2 * i
                pltpu.make_async_copy(y_hbm.at[src_v.at[j]], rows0_v,
                                      sem0).wait()
                pltpu.async_copy(y_hbm.at[src_v.at[j + 1]], rows1_v, sem1)
                pltpu.sync_copy(rows0_v, acc_sp.at[dst_v.at[j]], add=True)
                jn = jnp.minimum(j + 2, nbh - 1)
                pltpu.make_async_copy(y_hbm.at[src_v.at[j + 1]], rows1_v,
                                      sem1).wait()
                pltpu.async_copy(y_hbm.at[src_v.at[jn]], rows0_v, sem0)
                pltpu.sync_copy(rows1_v, acc_sp.at[dst_v.at[j + 1]],
                                add=True)
                return carry

            lax.fori_loop(0, nbh // 2, body, 0)
            pltpu.make_async_copy(y_hbm.at[src_v.at[nbh - 1]], rows0_v,
                                  sem0).wait()
        plsc.subcore_barrier()
        pltpu.sync_copy(acc_sp.at[pl.ds(s * RPT, RPT)],
                        out_hbm.at[c, pl.ds(s * RPT, RPT)])

    return agg


_agg_hid = _make_agg(HID)
_agg_cls = _make_agg(N_CLS)


def kernel(x, edge_index, W0, b0, W1, b1):
    src = edge_index[0]
    dst = edge_index[1]
    pad = EPAD - E
    srcp = jnp.concatenate(
        [src, jnp.zeros((pad,), jnp.int32)]).reshape(NW, NB, B)
    dstp = jnp.concatenate(
        [dst, jnp.full((pad,), DUMMY, jnp.int32)]).reshape(NW, NB, B)

    z_deg = jnp.zeros((RPT,), jnp.float32)
    deg = _deg_kernel(dstp, z_deg)
    xw0 = _mm(x, W0)
    y0, dinv = _scale0(deg[:, :N].reshape(2, N, 1), xw0)

    z_hid = jnp.zeros((RPT, HID), jnp.float32)
    z_cls = jnp.zeros((RPT, N_CLS), jnp.float32)

    acc0 = _agg_hid(y0, srcp, dstp, z_hid)
    y1 = _mid(acc0[:, :N, :], y0, dinv,
              b0.reshape(1, HID), W1)
    acc1 = _agg_cls(y1, srcp, dstp, z_cls)
    out = _fin(acc1[:, :N, :], y1, dinv, b1.reshape(1, N_CLS))
    return out


# speedup vs baseline: 25.8191x; 2.1688x over previous
import functools

import jax
import jax.numpy as jnp
from jax import lax
from jax.experimental import pallas as pl
from jax.experimental.pallas import tpu as pltpu
from jax.experimental.pallas import tpu_sc as plsc

N = 10000
E = 320000
D_IN = 128
HID = 128
N_CLS = 64
F = 64

NPAD = 10240
NW = 32
NB = 80
B = 128
EPAD = NW * NB * B
RPT = NPAD // 16
SRT = N // 16
DUMMY = N + 16

MT = 1000


def _mm_body(x_ref, w_ref, o_ref):
    o_ref[...] = jnp.dot(x_ref[...], w_ref[...],
                         preferred_element_type=jnp.float32)


def _mm(x, w):
    m, k = x.shape
    n = w.shape[1]
    return pl.pallas_call(
        _mm_body,
        grid=(m // MT,),
        in_specs=[pl.BlockSpec((MT, k), lambda i: (i, 0)),
                  pl.BlockSpec((k, n), lambda i: (0, 0))],
        out_specs=pl.BlockSpec((MT, n), lambda i: (i, 0)),
        out_shape=jax.ShapeDtypeStruct((m, n), jnp.float32),
    )(x, w)


def _scale0_body(deg_ref, xw_ref, ya_ref, yb_ref, dinv_ref):
    d = deg_ref[0] + deg_ref[1] + 1.0
    dinv = lax.rsqrt(d)
    dinv_ref[...] = dinv
    y = xw_ref[...] * dinv
    ya_ref[...] = y[:, :F]
    yb_ref[...] = y[:, F:]


def _scale0(deg, xw):
    return pl.pallas_call(
        _scale0_body,
        grid=(N // MT,),
        in_specs=[pl.BlockSpec((2, MT, 1), lambda i: (0, i, 0)),
                  pl.BlockSpec((MT, HID), lambda i: (i, 0))],
        out_specs=[pl.BlockSpec((MT, F), lambda i: (i, 0)),
                   pl.BlockSpec((MT, F), lambda i: (i, 0)),
                   pl.BlockSpec((MT, 1), lambda i: (i, 0))],
        out_shape=[jax.ShapeDtypeStruct((N, F), jnp.float32),
                   jax.ShapeDtypeStruct((N, F), jnp.float32),
                   jax.ShapeDtypeStruct((N, 1), jnp.float32)],
    )(deg, xw)


def _mid_body(acca_ref, accb_ref, ya_ref, yb_ref, dinv_ref, b0_ref, w1_ref,
              y1_ref):
    dinv = dinv_ref[...]
    h_lo = acca_ref[0] + acca_ref[1] + ya_ref[...]
    h_hi = accb_ref[0] + accb_ref[1] + yb_ref[...]
    h = jnp.concatenate([h_lo, h_hi], axis=1)
    h = jnp.maximum(dinv * h + b0_ref[...], 0.0)
    y1_ref[...] = jnp.dot(h, w1_ref[...],
                          preferred_element_type=jnp.float32) * dinv


def _mid(acca, accb, ya, yb, dinv, b0, w1):
    return pl.pallas_call(
        _mid_body,
        grid=(N // MT,),
        in_specs=[pl.BlockSpec((2, MT, F), lambda i: (0, i, 0)),
                  pl.BlockSpec((2, MT, F), lambda i: (0, i, 0)),
                  pl.BlockSpec((MT, F), lambda i: (i, 0)),
                  pl.BlockSpec((MT, F), lambda i: (i, 0)),
                  pl.BlockSpec((MT, 1), lambda i: (i, 0)),
                  pl.BlockSpec((1, HID), lambda i: (0, 0)),
                  pl.BlockSpec((HID, N_CLS), lambda i: (0, 0))],
        out_specs=pl.BlockSpec((MT, N_CLS), lambda i: (i, 0)),
        out_shape=jax.ShapeDtypeStruct((N, N_CLS), jnp.float32),
    )(acca, accb, ya, yb, dinv, b0, w1)


def _fin_body(acc_ref, y1_ref, dinv_ref, b1_ref, o_ref):
    o_ref[...] = (dinv_ref[...] * (acc_ref[0] + acc_ref[1] + y1_ref[...])
                  + b1_ref[...])


def _fin(acc, y1, dinv, b1):
    return pl.pallas_call(
        _fin_body,
        grid=(N // MT,),
        in_specs=[pl.BlockSpec((2, MT, N_CLS), lambda i: (0, i, 0)),
                  pl.BlockSpec((MT, N_CLS), lambda i: (i, 0)),
                  pl.BlockSpec((MT, 1), lambda i: (i, 0)),
                  pl.BlockSpec((1, N_CLS), lambda i: (0, 0))],
        out_specs=pl.BlockSpec((MT, N_CLS), lambda i: (i, 0)),
        out_shape=jax.ShapeDtypeStruct((N, N_CLS), jnp.float32),
    )(acc, y1, dinv, b1)


_MESH = plsc.VectorSubcoreMesh(core_axis_name="c", subcore_axis_name="s")
_SC_PARAMS = pltpu.CompilerParams(use_tc_tiling_on_sc=False)


@functools.partial(
    pl.kernel, mesh=_MESH,
    compiler_params=_SC_PARAMS,
    out_type=jax.ShapeDtypeStruct((2, NPAD), jnp.float32),
    scratch_types=[pltpu.VMEM((NB, B), jnp.int32),
                   pltpu.VMEM((B,), jnp.float32),
                   pltpu.VMEM_SHARED((NPAD,), jnp.float32)])
def _deg_kernel(dst_hbm, zeros_hbm, out_hbm, dst_v, ones_v, acc_sp):
    c = lax.axis_index("c")
    s = lax.axis_index("s")
    w = c * 16 + s
    pltpu.sync_copy(zeros_hbm, acc_sp.at[pl.ds(s * RPT, RPT)])
    pltpu.sync_copy(dst_hbm.at[w], dst_v)
    for i in range(B // 16):
        ones_v[pl.ds(i * 16, 16)] = jnp.ones((16,), jnp.float32)
    plsc.subcore_barrier()

    def body(j, carry):
        pltpu.sync_copy(ones_v, acc_sp.at[dst_v.at[j]], add=True)
        return carry

    lax.fori_loop(0, NB, body, 0)
    plsc.subcore_barrier()
    pltpu.sync_copy(acc_sp.at[pl.ds(s * RPT, RPT)],
                    out_hbm.at[c, pl.ds(s * RPT, RPT)])


def _agg_pass(y_hbm, out_hbm, s, c, src_v, dst_v, rows0_v, rows1_v,
              y_sp, acc_sp, sem0, sem1):
    pltpu.sync_copy(y_hbm.at[pl.ds(s * SRT, SRT)], y_sp.at[pl.ds(s * SRT, SRT)])
    plsc.subcore_barrier()

    pltpu.async_copy(y_sp.at[src_v.at[0]], rows0_v, sem0)

    def body(i, carry):
        j = 2 * i
        pltpu.make_async_copy(y_sp.at[src_v.at[j]], rows0_v, sem0).wait()
        pltpu.async_copy(y_sp.at[src_v.at[j + 1]], rows1_v, sem1)
        pltpu.sync_copy(rows0_v, acc_sp.at[dst_v.at[j]], add=True)
        jn = jnp.minimum(j + 2, NB - 1)
        pltpu.make_async_copy(y_sp.at[src_v.at[j + 1]], rows1_v, sem1).wait()
        pltpu.async_copy(y_sp.at[src_v.at[jn]], rows0_v, sem0)
        pltpu.sync_copy(rows1_v, acc_sp.at[dst_v.at[j + 1]], add=True)
        return carry

    lax.fori_loop(0, NB // 2, body, 0)
    pltpu.make_async_copy(y_sp.at[src_v.at[NB - 1]], rows0_v, sem0).wait()
    plsc.subcore_barrier()
    pltpu.sync_copy(acc_sp.at[pl.ds(s * RPT, RPT)],
                    out_hbm.at[c, pl.ds(s * RPT, RPT)])


def _make_agg(nparts):
    scratch = [pltpu.VMEM((NB, B), jnp.int32),
               pltpu.VMEM((NB, B), jnp.int32),
               pltpu.VMEM((B, F), jnp.float32),
               pltpu.VMEM((B, F), jnp.float32),
               pltpu.VMEM_SHARED((N, F), jnp.float32),
               pltpu.VMEM_SHARED((NPAD, F), jnp.float32),
               pltpu.SemaphoreType.DMA,
               pltpu.SemaphoreType.DMA]
    out_type = [jax.ShapeDtypeStruct((2, NPAD, F), jnp.float32)] * nparts

    @functools.partial(pl.kernel, mesh=_MESH, compiler_params=_SC_PARAMS,
                       out_type=out_type, scratch_types=scratch)
    def agg(*refs):
        y_hbms = refs[:nparts]
        src_hbm, dst_hbm, zeros_hbm = refs[nparts:nparts + 3]
        out_hbms = refs[nparts + 3:2 * nparts + 3]
        (src_v, dst_v, rows0_v, rows1_v, y_sp, acc_sp,
         sem0, sem1) = refs[2 * nparts + 3:]
        c = lax.axis_index("c")
        s = lax.axis_index("s")
        w = c * 16 + s
        pltpu.sync_copy(src_hbm.at[w], src_v)
        pltpu.sync_copy(dst_hbm.at[w], dst_v)
        for p in range(nparts):
            pltpu.sync_copy(zeros_hbm, acc_sp.at[pl.ds(s * RPT, RPT)])
            _agg_pass(y_hbms[p], out_hbms[p], s, c, src_v, dst_v,
                      rows0_v, rows1_v, y_sp, acc_sp, sem0, sem1)

    return agg


_agg2 = _make_agg(2)
_agg1 = _make_agg(1)


def kernel(x, edge_index, W0, b0, W1, b1):
    src = edge_index[0]
    dst = edge_index[1]
    pad = EPAD - E
    srcp = jnp.concatenate(
        [src, jnp.zeros((pad,), jnp.int32)]).reshape(NW, NB, B)
    dstp = jnp.concatenate(
        [dst, jnp.full((pad,), DUMMY, jnp.int32)]).reshape(NW, NB, B)

    z_deg = jnp.zeros((RPT,), jnp.float32)
    deg = _deg_kernel(dstp, z_deg)
    xw0 = _mm(x, W0)
    ya, yb, dinv = _scale0(deg[:, :N].reshape(2, N, 1), xw0)

    z_acc = jnp.zeros((RPT, F), jnp.float32)
    acca, accb = _agg2(ya, yb, srcp, dstp, z_acc)
    y1 = _mid(acca[:, :N, :], accb[:, :N, :], ya, yb, dinv,
              b0.reshape(1, HID), W1)
    acc1, = _agg1(y1, srcp, dstp, z_acc)
    out = _fin(acc1[:, :N, :], y1, dinv, b1.reshape(1, N_CLS))
    return out


# speedup vs baseline: 26.8614x; 1.0404x over previous
import functools

import jax
import jax.numpy as jnp
from jax import lax
from jax.experimental import pallas as pl
from jax.experimental.pallas import tpu as pltpu
from jax.experimental.pallas import tpu_sc as plsc

N = 10000
E = 320000
D_IN = 128
HID = 128
N_CLS = 64
F = 64

NPAD = 10240
NW = 32
NB = 80
B = 128
EPAD = NW * NB * B
RPT = NPAD // 16
SRT = N // 16
DUMMY = N + 16

MT = 1000


def _mm_body(x_ref, w_ref, o_ref):
    o_ref[...] = jnp.dot(x_ref[...], w_ref[...],
                         preferred_element_type=jnp.float32)


def _mm(x, w):
    m, k = x.shape
    n = w.shape[1]
    return pl.pallas_call(
        _mm_body,
        grid=(m // MT,),
        in_specs=[pl.BlockSpec((MT, k), lambda i: (i, 0)),
                  pl.BlockSpec((k, n), lambda i: (0, 0))],
        out_specs=pl.BlockSpec((MT, n), lambda i: (i, 0)),
        out_shape=jax.ShapeDtypeStruct((m, n), jnp.float32),
    )(x, w)


def _scale0_body(deg_ref, xw_ref, ya_ref, yb_ref, dinv_ref):
    d = deg_ref[0] + deg_ref[1] + 1.0
    dinv = lax.rsqrt(d)
    dinv_ref[...] = dinv
    y = xw_ref[...] * dinv
    ya_ref[...] = y[:, :F]
    yb_ref[...] = y[:, F:]


def _scale0(deg, xw):
    return pl.pallas_call(
        _scale0_body,
        grid=(N // MT,),
        in_specs=[pl.BlockSpec((2, MT, 1), lambda i: (0, i, 0)),
                  pl.BlockSpec((MT, HID), lambda i: (i, 0))],
        out_specs=[pl.BlockSpec((MT, F), lambda i: (i, 0)),
                   pl.BlockSpec((MT, F), lambda i: (i, 0)),
                   pl.BlockSpec((MT, 1), lambda i: (i, 0))],
        out_shape=[jax.ShapeDtypeStruct((N, F), jnp.float32),
                   jax.ShapeDtypeStruct((N, F), jnp.float32),
                   jax.ShapeDtypeStruct((N, 1), jnp.float32)],
    )(deg, xw)


def _mid_body(acca_ref, accb_ref, ya_ref, yb_ref, dinv_ref, b0_ref, w1_ref,
              y1_ref):
    dinv = dinv_ref[...]
    h_lo = acca_ref[0] + acca_ref[1] + ya_ref[...]
    h_hi = accb_ref[0] + accb_ref[1] + yb_ref[...]
    h = jnp.concatenate([h_lo, h_hi], axis=1)
    h = jnp.maximum(dinv * h + b0_ref[...], 0.0)
    y1_ref[...] = jnp.dot(h, w1_ref[...],
                          preferred_element_type=jnp.float32) * dinv


def _mid(acca, accb, ya, yb, dinv, b0, w1):
    return pl.pallas_call(
        _mid_body,
        grid=(N // MT,),
        in_specs=[pl.BlockSpec((2, MT, F), lambda i: (0, i, 0)),
                  pl.BlockSpec((2, MT, F), lambda i: (0, i, 0)),
                  pl.BlockSpec((MT, F), lambda i: (i, 0)),
                  pl.BlockSpec((MT, F), lambda i: (i, 0)),
                  pl.BlockSpec((MT, 1), lambda i: (i, 0)),
                  pl.BlockSpec((1, HID), lambda i: (0, 0)),
                  pl.BlockSpec((HID, N_CLS), lambda i: (0, 0))],
        out_specs=pl.BlockSpec((MT, N_CLS), lambda i: (i, 0)),
        out_shape=jax.ShapeDtypeStruct((N, N_CLS), jnp.float32),
    )(acca, accb, ya, yb, dinv, b0, w1)


def _fin_body(acc_ref, y1_ref, dinv_ref, b1_ref, o_ref):
    o_ref[...] = (dinv_ref[...] * (acc_ref[0] + acc_ref[1] + y1_ref[...])
                  + b1_ref[...])


def _fin(acc, y1, dinv, b1):
    return pl.pallas_call(
        _fin_body,
        grid=(N // MT,),
        in_specs=[pl.BlockSpec((2, MT, N_CLS), lambda i: (0, i, 0)),
                  pl.BlockSpec((MT, N_CLS), lambda i: (i, 0)),
                  pl.BlockSpec((MT, 1), lambda i: (i, 0)),
                  pl.BlockSpec((1, N_CLS), lambda i: (0, 0))],
        out_specs=pl.BlockSpec((MT, N_CLS), lambda i: (i, 0)),
        out_shape=jax.ShapeDtypeStruct((N, N_CLS), jnp.float32),
    )(acc, y1, dinv, b1)


_MESH = plsc.VectorSubcoreMesh(core_axis_name="c", subcore_axis_name="s")
_SC_PARAMS = pltpu.CompilerParams(use_tc_tiling_on_sc=False)


@functools.partial(
    pl.kernel, mesh=_MESH,
    compiler_params=_SC_PARAMS,
    out_type=jax.ShapeDtypeStruct((2, NPAD), jnp.float32),
    scratch_types=[pltpu.VMEM((NB, B), jnp.int32),
                   pltpu.VMEM((B,), jnp.float32),
                   pltpu.VMEM_SHARED((NPAD,), jnp.float32)])
def _deg_kernel(dst_hbm, zeros_hbm, out_hbm, dst_v, ones_v, acc_sp):
    c = lax.axis_index("c")
    s = lax.axis_index("s")
    w = c * 16 + s
    pltpu.sync_copy(zeros_hbm, acc_sp.at[pl.ds(s * RPT, RPT)])
    pltpu.sync_copy(dst_hbm.at[w], dst_v)
    for i in range(B // 16):
        ones_v[pl.ds(i * 16, 16)] = jnp.ones((16,), jnp.float32)
    plsc.subcore_barrier()

    def body(j, carry):
        pltpu.sync_copy(ones_v, acc_sp.at[dst_v.at[j]], add=True)
        return carry

    lax.fori_loop(0, NB, body, 0)
    plsc.subcore_barrier()
    pltpu.sync_copy(acc_sp.at[pl.ds(s * RPT, RPT)],
                    out_hbm.at[c, pl.ds(s * RPT, RPT)])


def _agg_pass(y_hbm, out_hbm, s, c, src_v, dst_v, rows0_v, rows1_v,
              y_sp, acc_sp, sem0, sem1):
    pltpu.sync_copy(y_hbm.at[pl.ds(s * SRT, SRT)], y_sp.at[pl.ds(s * SRT, SRT)])
    plsc.subcore_barrier()

    pltpu.async_copy(y_sp.at[src_v.at[0]], rows0_v, sem0)

    def body(i, carry):
        j = 2 * i
        pltpu.make_async_copy(y_sp.at[src_v.at[j]], rows0_v, sem0).wait()
        pltpu.async_copy(y_sp.at[src_v.at[j + 1]], rows1_v, sem1)
        pltpu.sync_copy(rows0_v, acc_sp.at[dst_v.at[j]], add=True)
        jn = jnp.minimum(j + 2, NB - 1)
        pltpu.make_async_copy(y_sp.at[src_v.at[j + 1]], rows1_v, sem1).wait()
        pltpu.async_copy(y_sp.at[src_v.at[jn]], rows0_v, sem0)
        pltpu.sync_copy(rows1_v, acc_sp.at[dst_v.at[j + 1]], add=True)
        return carry

    lax.fori_loop(0, NB // 2, body, 0)
    pltpu.make_async_copy(y_sp.at[src_v.at[NB - 1]], rows0_v, sem0).wait()
    plsc.subcore_barrier()
    pltpu.sync_copy(acc_sp.at[pl.ds(s * RPT, RPT)],
                    out_hbm.at[c, pl.ds(s * RPT, RPT)])


def _make_agg(nparts):
    scratch = [pltpu.VMEM((NB, B), jnp.int32),
               pltpu.VMEM((NB, B), jnp.int32),
               pltpu.VMEM((B, F), jnp.float32),
               pltpu.VMEM((B, F), jnp.float32),
               pltpu.VMEM_SHARED((N, F), jnp.float32),
               pltpu.VMEM_SHARED((NPAD, F), jnp.float32),
               pltpu.SemaphoreType.DMA,
               pltpu.SemaphoreType.DMA]
    out_type = [jax.ShapeDtypeStruct((2, NPAD, F), jnp.float32)] * nparts

    @functools.partial(pl.kernel, mesh=_MESH, compiler_params=_SC_PARAMS,
                       out_type=out_type, scratch_types=scratch)
    def agg(*refs):
        y_hbms = refs[:nparts]
        src_hbm, dst_hbm, zeros_hbm = refs[nparts:nparts + 3]
        out_hbms = refs[nparts + 3:2 * nparts + 3]
        (src_v, dst_v, rows0_v, rows1_v, y_sp, acc_sp,
         sem0, sem1) = refs[2 * nparts + 3:]
        c = lax.axis_index("c")
        s = lax.axis_index("s")
        w = c * 16 + s
        pltpu.sync_copy(src_hbm.at[w], src_v)
        pltpu.sync_copy(dst_hbm.at[w], dst_v)
        for p in range(nparts):
            pltpu.sync_copy(zeros_hbm, acc_sp.at[pl.ds(s * RPT, RPT)])
            _agg_pass(y_hbms[p], out_hbms[p], s, c, src_v, dst_v,
                      rows0_v, rows1_v, y_sp, acc_sp, sem0, sem1)

    return agg


_agg2 = _make_agg(2)
_agg1 = _make_agg(1)


def kernel(x, edge_index, W0, b0, W1, b1):
    src = edge_index[0]
    dst = edge_index[1]
    pad = EPAD - E
    srcp = jnp.concatenate(
        [src, jnp.zeros((pad,), jnp.int32)]).reshape(NW, NB, B)
    dstp = jnp.concatenate(
        [dst, jnp.full((pad,), DUMMY, jnp.int32)]).reshape(NW, NB, B)

    z_deg = jnp.zeros((RPT,), jnp.float32)
    deg = _deg_kernel(dstp, z_deg)
    xw0 = _mm(x, W0)
    ya, yb, dinv = _scale0(deg.reshape(2, NPAD, 1), xw0)

    z_acc = jnp.zeros((RPT, F), jnp.float32)
    acca, accb = _agg2(ya, yb, srcp, dstp, z_acc)
    y1 = _mid(acca, accb, ya, yb, dinv,
              b0.reshape(1, HID), W1)
    acc1, = _agg1(y1, srcp, dstp, z_acc)
    out = _fin(acc1, y1, dinv, b1.reshape(1, N_CLS))
    return out


# speedup vs baseline: 30.0312x; 1.1180x over previous
import functools

import jax
import jax.numpy as jnp
from jax import lax
from jax.experimental import pallas as pl
from jax.experimental.pallas import tpu as pltpu
from jax.experimental.pallas import tpu_sc as plsc

N = 10000
E = 320000
D_IN = 128
HID = 128
N_CLS = 64
F = 64

NPAD = 10240
NW = 32
NB = 80
B = 128
EPAD = NW * NB * B
RPT = NPAD // 16
SRT = N // 16
DUMMY = N + 16

MT = 1000


def _mm_body(x_ref, w_ref, o_ref):
    o_ref[...] = jnp.dot(x_ref[...], w_ref[...],
                         preferred_element_type=jnp.float32)


def _mm(x, w):
    m, k = x.shape
    n = w.shape[1]
    return pl.pallas_call(
        _mm_body,
        grid=(m // MT,),
        in_specs=[pl.BlockSpec((MT, k), lambda i: (i, 0)),
                  pl.BlockSpec((k, n), lambda i: (0, 0))],
        out_specs=pl.BlockSpec((MT, n), lambda i: (i, 0)),
        out_shape=jax.ShapeDtypeStruct((m, n), jnp.float32),
    )(x, w)


def _scale0_body(deg_ref, xw_ref, y_ref, dinv_ref):
    d = deg_ref[0] + deg_ref[1] + 1.0
    dinv = lax.rsqrt(d)
    dinv_ref[...] = dinv
    y_ref[...] = xw_ref[...] * dinv


def _scale0(deg, xw):
    return pl.pallas_call(
        _scale0_body,
        grid=(N // MT,),
        in_specs=[pl.BlockSpec((2, MT, 1), lambda i: (0, i, 0)),
                  pl.BlockSpec((MT, HID), lambda i: (i, 0))],
        out_specs=[pl.BlockSpec((MT, HID), lambda i: (i, 0)),
                   pl.BlockSpec((MT, 1), lambda i: (i, 0))],
        out_shape=[jax.ShapeDtypeStruct((N, HID), jnp.float32),
                   jax.ShapeDtypeStruct((N, 1), jnp.float32)],
    )(deg, xw)


def _mid_body(acc_ref, y0_ref, dinv_ref, b0_ref, w1_ref, y1_ref):
    dinv = dinv_ref[...]
    h = acc_ref[0] + acc_ref[1] + y0_ref[...]
    h = jnp.maximum(dinv * h + b0_ref[...], 0.0)
    y1 = jnp.dot(h, w1_ref[...],
                 preferred_element_type=jnp.float32) * dinv
    y1_ref[...] = jnp.concatenate(
        [y1, jnp.zeros((MT, HID - N_CLS), jnp.float32)], axis=1)


def _mid(acc, y0, dinv, b0, w1):
    return pl.pallas_call(
        _mid_body,
        grid=(N // MT,),
        in_specs=[pl.BlockSpec((2, MT, HID), lambda i: (0, i, 0)),
                  pl.BlockSpec((MT, HID), lambda i: (i, 0)),
                  pl.BlockSpec((MT, 1), lambda i: (i, 0)),
                  pl.BlockSpec((1, HID), lambda i: (0, 0)),
                  pl.BlockSpec((HID, N_CLS), lambda i: (0, 0))],
        out_specs=pl.BlockSpec((MT, HID), lambda i: (i, 0)),
        out_shape=jax.ShapeDtypeStruct((N, HID), jnp.float32),
    )(acc, y0, dinv, b0, w1)


def _fin_body(acc_ref, y1_ref, dinv_ref, b1_ref, o_ref):
    agg = acc_ref[0, :, :N_CLS] + acc_ref[1, :, :N_CLS]
    o_ref[...] = (dinv_ref[...] * (agg + y1_ref[:, :N_CLS]) + b1_ref[...])


def _fin(acc, y1, dinv, b1):
    return pl.pallas_call(
        _fin_body,
        grid=(N // MT,),
        in_specs=[pl.BlockSpec((2, MT, HID), lambda i: (0, i, 0)),
                  pl.BlockSpec((MT, HID), lambda i: (i, 0)),
                  pl.BlockSpec((MT, 1), lambda i: (i, 0)),
                  pl.BlockSpec((1, N_CLS), lambda i: (0, 0))],
        out_specs=pl.BlockSpec((MT, N_CLS), lambda i: (i, 0)),
        out_shape=jax.ShapeDtypeStruct((N, N_CLS), jnp.float32),
    )(acc, y1, dinv, b1)


_MESH = plsc.VectorSubcoreMesh(core_axis_name="c", subcore_axis_name="s")
_SC_PARAMS = pltpu.CompilerParams(use_tc_tiling_on_sc=False)


@functools.partial(
    pl.kernel, mesh=_MESH,
    compiler_params=_SC_PARAMS,
    out_type=jax.ShapeDtypeStruct((2, NPAD), jnp.float32),
    scratch_types=[pltpu.VMEM((NB, B), jnp.int32),
                   pltpu.VMEM((B,), jnp.float32),
                   pltpu.VMEM_SHARED((NPAD,), jnp.float32)])
def _deg_kernel(dst_hbm, zeros_hbm, out_hbm, dst_v, ones_v, acc_sp):
    c = lax.axis_index("c")
    s = lax.axis_index("s")
    w = c * 16 + s
    pltpu.sync_copy(zeros_hbm, acc_sp.at[pl.ds(s * RPT, RPT)])
    pltpu.sync_copy(dst_hbm.at[w], dst_v)
    for i in range(B // 16):
        ones_v[pl.ds(i * 16, 16)] = jnp.ones((16,), jnp.float32)
    plsc.subcore_barrier()

    def body(j, carry):
        pltpu.sync_copy(ones_v, acc_sp.at[dst_v.at[j]], add=True)
        return carry

    lax.fori_loop(0, NB, body, 0)
    plsc.subcore_barrier()
    pltpu.sync_copy(acc_sp.at[pl.ds(s * RPT, RPT)],
                    out_hbm.at[c, pl.ds(s * RPT, RPT)])


def _agg_pass(y_hbm, out_hbm, p, s, c, src_v, dst_v, rows0_v, rows1_v,
              y_sp, acc_sp, sem0, sem1):
    pltpu.sync_copy(y_hbm.at[pl.ds(s * SRT, SRT), pl.ds(F * p, F)],
                    y_sp.at[pl.ds(s * SRT, SRT)])
    plsc.subcore_barrier()

    pltpu.async_copy(y_sp.at[src_v.at[0]], rows0_v, sem0)

    def body(i, carry):
        j = 2 * i
        pltpu.make_async_copy(y_sp.at[src_v.at[j]], rows0_v, sem0).wait()
        pltpu.async_copy(y_sp.at[src_v.at[j + 1]], rows1_v, sem1)
        pltpu.sync_copy(rows0_v, acc_sp.at[dst_v.at[j]], add=True)
        jn = jnp.minimum(j + 2, NB - 1)
        pltpu.make_async_copy(y_sp.at[src_v.at[j + 1]], rows1_v, sem1).wait()
        pltpu.async_copy(y_sp.at[src_v.at[jn]], rows0_v, sem0)
        pltpu.sync_copy(rows1_v, acc_sp.at[dst_v.at[j + 1]], add=True)
        return carry

    lax.fori_loop(0, NB // 2, body, 0)
    pltpu.make_async_copy(y_sp.at[src_v.at[NB - 1]], rows0_v, sem0).wait()
    plsc.subcore_barrier()
    pltpu.sync_copy(acc_sp.at[pl.ds(s * RPT, RPT)],
                    out_hbm.at[c, pl.ds(s * RPT, RPT), pl.ds(F * p, F)])


def _make_agg(nparts):
    scratch = [pltpu.VMEM((NB, B), jnp.int32),
               pltpu.VMEM((NB, B), jnp.int32),
               pltpu.VMEM((B, F), jnp.float32),
               pltpu.VMEM((B, F), jnp.float32),
               pltpu.VMEM_SHARED((N, F), jnp.float32),
               pltpu.VMEM_SHARED((NPAD, F), jnp.float32),
               pltpu.SemaphoreType.DMA,
               pltpu.SemaphoreType.DMA]
    out_type = jax.ShapeDtypeStruct((2, NPAD, HID), jnp.float32)

    @functools.partial(pl.kernel, mesh=_MESH, compiler_params=_SC_PARAMS,
                       out_type=out_type, scratch_types=scratch)
    def agg(y_hbm, src_hbm, dst_hbm, zeros_hbm, out_hbm,
            src_v, dst_v, rows0_v, rows1_v, y_sp, acc_sp, sem0, sem1):
        c = lax.axis_index("c")
        s = lax.axis_index("s")
        w = c * 16 + s
        pltpu.sync_copy(src_hbm.at[w], src_v)
        pltpu.sync_copy(dst_hbm.at[w], dst_v)
        for p in range(nparts):
            pltpu.sync_copy(zeros_hbm, acc_sp.at[pl.ds(s * RPT, RPT)])
            _agg_pass(y_hbm, out_hbm, p, s, c, src_v, dst_v,
                      rows0_v, rows1_v, y_sp, acc_sp, sem0, sem1)

    return agg


_agg2 = _make_agg(2)
_agg1 = _make_agg(1)


def kernel(x, edge_index, W0, b0, W1, b1):
    src = edge_index[0]
    dst = edge_index[1]
    pad = EPAD - E
    srcp = jnp.concatenate(
        [src, jnp.zeros((pad,), jnp.int32)]).reshape(NW, NB, B)
    dstp = jnp.concatenate(
        [dst, jnp.full((pad,), DUMMY, jnp.int32)]).reshape(NW, NB, B)

    z_deg = jnp.zeros((RPT,), jnp.float32)
    deg = _deg_kernel(dstp, z_deg)
    xw0 = _mm(x, W0)
    y0, dinv = _scale0(deg.reshape(2, NPAD, 1), xw0)

    z_acc = jnp.zeros((RPT, F), jnp.float32)
    acc0 = _agg2(y0, srcp, dstp, z_acc)
    y1 = _mid(acc0, y0, dinv, b0.reshape(1, HID), W1)
    acc1 = _agg1(y1, srcp, dstp, z_acc)
    out = _fin(acc1, y1, dinv, b1.reshape(1, N_CLS))
    return out


# speedup vs baseline: 32.8351x; 1.0934x over previous
import functools

import jax
import jax.numpy as jnp
from jax import lax
from jax.experimental import pallas as pl
from jax.experimental.pallas import tpu as pltpu
from jax.experimental.pallas import tpu_sc as plsc

N = 10000
E = 320000
D_IN = 128
HID = 128
N_CLS = 64
F = 64

NPAD = 10240
NW = 32
NB = 80
B = 128
EPAD = NW * NB * B
RPT = NPAD // 16
SRT = N // 16
DUMMY = N + 16

MT = 1000


def _mm_body(x_ref, w_ref, o_ref):
    o_ref[...] = jnp.dot(x_ref[...], w_ref[...],
                         preferred_element_type=jnp.float32)


def _mm(x, w):
    m, k = x.shape
    n = w.shape[1]
    return pl.pallas_call(
        _mm_body,
        grid=(m // MT,),
        in_specs=[pl.BlockSpec((MT, k), lambda i: (i, 0)),
                  pl.BlockSpec((k, n), lambda i: (0, 0))],
        out_specs=pl.BlockSpec((MT, n), lambda i: (i, 0)),
        out_shape=jax.ShapeDtypeStruct((m, n), jnp.float32),
    )(x, w)


def _scale0_body(deg_ref, xw_ref, y_ref, dinv_ref):
    d = deg_ref[0] + deg_ref[1] + 1.0
    dinv = lax.rsqrt(d)
    dinv_ref[...] = dinv
    y_ref[...] = xw_ref[...] * dinv


def _scale0(deg, xw):
    return pl.pallas_call(
        _scale0_body,
        grid=(N // MT,),
        in_specs=[pl.BlockSpec((2, MT, 1), lambda i: (0, i, 0)),
                  pl.BlockSpec((MT, HID), lambda i: (i, 0))],
        out_specs=[pl.BlockSpec((MT, HID), lambda i: (i, 0)),
                   pl.BlockSpec((MT, 1), lambda i: (i, 0))],
        out_shape=[jax.ShapeDtypeStruct((N, HID), jnp.float32),
                   jax.ShapeDtypeStruct((N, 1), jnp.float32)],
    )(deg, xw)


def _mid_body(acc_ref, y0_ref, dinv_ref, b0_ref, w1_ref, y1_ref):
    dinv = dinv_ref[...]
    h = acc_ref[0] + acc_ref[1] + y0_ref[...]
    h = jnp.maximum(dinv * h + b0_ref[...], 0.0)
    y1 = jnp.dot(h, w1_ref[...],
                 preferred_element_type=jnp.float32) * dinv
    y1_ref[...] = jnp.concatenate(
        [y1, jnp.zeros((MT, HID - N_CLS), jnp.float32)], axis=1)


def _mid(acc, y0, dinv, b0, w1):
    return pl.pallas_call(
        _mid_body,
        grid=(N // MT,),
        in_specs=[pl.BlockSpec((2, MT, HID), lambda i: (0, i, 0)),
                  pl.BlockSpec((MT, HID), lambda i: (i, 0)),
                  pl.BlockSpec((MT, 1), lambda i: (i, 0)),
                  pl.BlockSpec((1, HID), lambda i: (0, 0)),
                  pl.BlockSpec((HID, N_CLS), lambda i: (0, 0))],
        out_specs=pl.BlockSpec((MT, HID), lambda i: (i, 0)),
        out_shape=jax.ShapeDtypeStruct((N, HID), jnp.float32),
    )(acc, y0, dinv, b0, w1)


def _fin_body(acc_ref, y1_ref, dinv_ref, b1_ref, o_ref):
    agg = acc_ref[0, :, :N_CLS] + acc_ref[1, :, :N_CLS]
    o_ref[...] = (dinv_ref[...] * (agg + y1_ref[:, :N_CLS]) + b1_ref[...])


def _fin(acc, y1, dinv, b1):
    return pl.pallas_call(
        _fin_body,
        grid=(N // MT,),
        in_specs=[pl.BlockSpec((2, MT, HID), lambda i: (0, i, 0)),
                  pl.BlockSpec((MT, HID), lambda i: (i, 0)),
                  pl.BlockSpec((MT, 1), lambda i: (i, 0)),
                  pl.BlockSpec((1, N_CLS), lambda i: (0, 0))],
        out_specs=pl.BlockSpec((MT, N_CLS), lambda i: (i, 0)),
        out_shape=jax.ShapeDtypeStruct((N, N_CLS), jnp.float32),
    )(acc, y1, dinv, b1)


_MESH = plsc.VectorSubcoreMesh(core_axis_name="c", subcore_axis_name="s")
_SC_PARAMS = pltpu.CompilerParams(use_tc_tiling_on_sc=False)


@functools.partial(
    pl.kernel, mesh=_MESH,
    compiler_params=_SC_PARAMS,
    out_type=jax.ShapeDtypeStruct((2, NPAD), jnp.float32),
    scratch_types=[pltpu.VMEM((NB, B), jnp.int32),
                   pltpu.VMEM((B,), jnp.float32),
                   pltpu.VMEM_SHARED((NPAD,), jnp.float32)])
def _deg_kernel(dst_hbm, zeros_hbm, out_hbm, dst_v, ones_v, acc_sp):
    c = lax.axis_index("c")
    s = lax.axis_index("s")
    w = c * 16 + s
    pltpu.sync_copy(zeros_hbm, acc_sp.at[pl.ds(s * RPT, RPT)])
    pltpu.sync_copy(dst_hbm.at[w], dst_v)
    for i in range(B // 16):
        ones_v[pl.ds(i * 16, 16)] = jnp.ones((16,), jnp.float32)
    plsc.subcore_barrier()

    def body(j, carry):
        pltpu.sync_copy(ones_v, acc_sp.at[dst_v.at[j]], add=True)
        return carry

    lax.fori_loop(0, NB, body, 0)
    plsc.subcore_barrier()
    pltpu.sync_copy(acc_sp.at[pl.ds(s * RPT, RPT)],
                    out_hbm.at[c, pl.ds(s * RPT, RPT)])


def _agg_pass(y_hbm, out_hbm, p, s, c, w, src_hbm, dst_hbm, src_v, dst_v,
              rows, gsem, ssem, y_sp, acc_sp):
    pltpu.sync_copy(y_hbm.at[pl.ds(s * SRT, SRT), pl.ds(F * p, F)],
                    y_sp.at[pl.ds(s * SRT, SRT)])
    plsc.subcore_barrier()

    def g(jb, k):
        pltpu.async_copy(y_sp.at[src_v.at[jb]], rows[k], gsem[k])

    def gwait(k):
        pltpu.make_async_copy(y_sp.at[src_v.at[0]], rows[k], gsem[k]).wait()

    def sc(jb, k):
        pltpu.async_copy(rows[k], acc_sp.at[dst_v.at[jb]], ssem[k],
                         add=True)

    def scwait(k):
        pltpu.make_async_copy(rows[k], acc_sp.at[dst_v.at[0]],
                              ssem[k]).wait()

    nbh = NB // 2
    for h in range(2):
        pltpu.sync_copy(src_hbm.at[w, pl.ds(h * nbh, nbh)], src_v)
        pltpu.sync_copy(dst_hbm.at[w, pl.ds(h * nbh, nbh)], dst_v)
        g(0, 0)
        g(1, 1)
        gwait(0); sc(0, 0); g(2, 2)
        gwait(1); sc(1, 1); g(3, 3)
        gwait(2); sc(2, 2); scwait(0); g(4, 0)
        gwait(3); sc(3, 3); scwait(1); g(5, 1)

        def body(i, carry):
            j0 = 4 * i
            gwait(0); sc(j0, 0); scwait(2); g(jnp.minimum(j0 + 2, nbh - 1), 2)
            gwait(1); sc(j0 + 1, 1); scwait(3)
            g(jnp.minimum(j0 + 3, nbh - 1), 3)
            gwait(2); sc(j0 + 2, 2); scwait(0)
            g(jnp.minimum(j0 + 4, nbh - 1), 0)
            gwait(3); sc(j0 + 3, 3); scwait(1)
            g(jnp.minimum(j0 + 5, nbh - 1), 1)
            return carry

        lax.fori_loop(1, nbh // 4, body, 0)
        gwait(0)
        gwait(1)
        scwait(2)
        scwait(3)

    plsc.subcore_barrier()
    pltpu.sync_copy(acc_sp.at[pl.ds(s * RPT, RPT)],
                    out_hbm.at[c, pl.ds(s * RPT, RPT), pl.ds(F * p, F)])


def _make_agg(nparts):
    scratch = [pltpu.VMEM((NB // 2, B), jnp.int32),
               pltpu.VMEM((NB // 2, B), jnp.int32),
               pltpu.VMEM((B, F), jnp.float32),
               pltpu.VMEM((B, F), jnp.float32),
               pltpu.VMEM((B, F), jnp.float32),
               pltpu.VMEM((B, F), jnp.float32),
               pltpu.VMEM_SHARED((N, F), jnp.float32),
               pltpu.VMEM_SHARED((NPAD, F), jnp.float32),
               pltpu.SemaphoreType.DMA,
               pltpu.SemaphoreType.DMA,
               pltpu.SemaphoreType.DMA,
               pltpu.SemaphoreType.DMA,
               pltpu.SemaphoreType.DMA,
               pltpu.SemaphoreType.DMA,
               pltpu.SemaphoreType.DMA,
               pltpu.SemaphoreType.DMA]
    out_type = jax.ShapeDtypeStruct((2, NPAD, HID), jnp.float32)

    @functools.partial(pl.kernel, mesh=_MESH, compiler_params=_SC_PARAMS,
                       out_type=out_type, scratch_types=scratch)
    def agg(y_hbm, src_hbm, dst_hbm, zeros_hbm, out_hbm,
            src_v, dst_v, r0, r1, r2, r3, y_sp, acc_sp,
            g0, g1, g2, g3, s0, s1, s2, s3):
        c = lax.axis_index("c")
        s = lax.axis_index("s")
        w = c * 16 + s
        rows = [r0, r1, r2, r3]
        gsem = [g0, g1, g2, g3]
        ssem = [s0, s1, s2, s3]
        for p in range(nparts):
            pltpu.sync_copy(zeros_hbm, acc_sp.at[pl.ds(s * RPT, RPT)])
            _agg_pass(y_hbm, out_hbm, p, s, c, w, src_hbm, dst_hbm,
                      src_v, dst_v, rows, gsem, ssem, y_sp, acc_sp)

    return agg


_agg2 = _make_agg(2)
_agg1 = _make_agg(1)


def kernel(x, edge_index, W0, b0, W1, b1):
    src = edge_index[0]
    dst = edge_index[1]
    pad = EPAD - E
    srcp = jnp.concatenate(
        [src, jnp.zeros((pad,), jnp.int32)]).reshape(NW, NB, B)
    dstp = jnp.concatenate(
        [dst, jnp.full((pad,), DUMMY, jnp.int32)]).reshape(NW, NB, B)

    z_deg = jnp.zeros((RPT,), jnp.float32)
    deg = _deg_kernel(dstp, z_deg)
    xw0 = _mm(x, W0)
    y0, dinv = _scale0(deg.reshape(2, NPAD, 1), xw0)

    z_acc = jnp.zeros((RPT, F), jnp.float32)
    acc0 = _agg2(y0, srcp, dstp, z_acc)
    y1 = _mid(acc0, y0, dinv, b0.reshape(1, HID), W1)
    acc1 = _agg1(y1, srcp, dstp, z_acc)
    out = _fin(acc1, y1, dinv, b1.reshape(1, N_CLS))
    return out


# speedup vs baseline: 33.2820x; 1.0136x over previous
import functools

import jax
import jax.numpy as jnp
from jax import lax
from jax.experimental import pallas as pl
from jax.experimental.pallas import tpu as pltpu
from jax.experimental.pallas import tpu_sc as plsc

N = 10000
E = 320000
D_IN = 128
HID = 128
N_CLS = 64
F = 64

NPAD = 10240
NW = 32
NB = 80
B = 128
EPAD = NW * NB * B
RPT = NPAD // 16
SRT = N // 16
DUMMY = N + 16

MT = 1000


def _mm_body(x_ref, w_ref, o_ref):
    o_ref[...] = jnp.dot(x_ref[...], w_ref[...],
                         preferred_element_type=jnp.float32)


def _mm(x, w):
    m, k = x.shape
    n = w.shape[1]
    return pl.pallas_call(
        _mm_body,
        grid=(m // MT,),
        in_specs=[pl.BlockSpec((MT, k), lambda i: (i, 0)),
                  pl.BlockSpec((k, n), lambda i: (0, 0))],
        out_specs=pl.BlockSpec((MT, n), lambda i: (i, 0)),
        out_shape=jax.ShapeDtypeStruct((m, n), jnp.float32),
    )(x, w)


def _scale0_body(deg_ref, xw_ref, y_ref, dinv_ref):
    d = deg_ref[0] + deg_ref[1] + 1.0
    dinv = lax.rsqrt(d)
    dinv_ref[...] = dinv
    y_ref[...] = xw_ref[...] * dinv


def _scale0(deg, xw):
    return pl.pallas_call(
        _scale0_body,
        grid=(N // MT,),
        in_specs=[pl.BlockSpec((2, MT, 1), lambda i: (0, i, 0)),
                  pl.BlockSpec((MT, HID), lambda i: (i, 0))],
        out_specs=[pl.BlockSpec((MT, HID), lambda i: (i, 0)),
                   pl.BlockSpec((MT, 1), lambda i: (i, 0))],
        out_shape=[jax.ShapeDtypeStruct((N, HID), jnp.float32),
                   jax.ShapeDtypeStruct((N, 1), jnp.float32)],
    )(deg, xw)


def _mid_body(acc_ref, y0_ref, dinv_ref, b0_ref, w1_ref, y1_ref):
    dinv = dinv_ref[...]
    h = acc_ref[0] + acc_ref[1] + y0_ref[...]
    h = jnp.maximum(dinv * h + b0_ref[...], 0.0)
    y1 = jnp.dot(h, w1_ref[...],
                 preferred_element_type=jnp.float32) * dinv
    y1_ref[...] = jnp.concatenate(
        [y1, jnp.zeros((MT, HID - N_CLS), jnp.float32)], axis=1)


def _mid(acc, y0, dinv, b0, w1):
    return pl.pallas_call(
        _mid_body,
        grid=(N // MT,),
        in_specs=[pl.BlockSpec((2, MT, HID), lambda i: (0, i, 0)),
                  pl.BlockSpec((MT, HID), lambda i: (i, 0)),
                  pl.BlockSpec((MT, 1), lambda i: (i, 0)),
                  pl.BlockSpec((1, HID), lambda i: (0, 0)),
                  pl.BlockSpec((HID, N_CLS), lambda i: (0, 0))],
        out_specs=pl.BlockSpec((MT, HID), lambda i: (i, 0)),
        out_shape=jax.ShapeDtypeStruct((N, HID), jnp.float32),
    )(acc, y0, dinv, b0, w1)


def _fin_body(acc_ref, y1_ref, dinv_ref, b1_ref, o_ref):
    agg = acc_ref[0, :, :N_CLS] + acc_ref[1, :, :N_CLS]
    o_ref[...] = (dinv_ref[...] * (agg + y1_ref[:, :N_CLS]) + b1_ref[...])


def _fin(acc, y1, dinv, b1):
    return pl.pallas_call(
        _fin_body,
        grid=(N // MT,),
        in_specs=[pl.BlockSpec((2, MT, HID), lambda i: (0, i, 0)),
                  pl.BlockSpec((MT, HID), lambda i: (i, 0)),
                  pl.BlockSpec((MT, 1), lambda i: (i, 0)),
                  pl.BlockSpec((1, N_CLS), lambda i: (0, 0))],
        out_specs=pl.BlockSpec((MT, N_CLS), lambda i: (i, 0)),
        out_shape=jax.ShapeDtypeStruct((N, N_CLS), jnp.float32),
    )(acc, y1, dinv, b1)


_MESH = plsc.VectorSubcoreMesh(core_axis_name="c", subcore_axis_name="s")
_SC_PARAMS = pltpu.CompilerParams(use_tc_tiling_on_sc=False)


@functools.partial(
    pl.kernel, mesh=_MESH,
    compiler_params=_SC_PARAMS,
    out_type=jax.ShapeDtypeStruct((2, NPAD), jnp.float32),
    scratch_types=[pltpu.VMEM((NB, B), jnp.int32),
                   pltpu.VMEM((B,), jnp.float32),
                   pltpu.VMEM_SHARED((NPAD,), jnp.float32)])
def _deg_kernel(dst_hbm, zeros_hbm, out_hbm, dst_v, ones_v, acc_sp):
    c = lax.axis_index("c")
    s = lax.axis_index("s")
    w = c * 16 + s
    pltpu.sync_copy(zeros_hbm, acc_sp.at[pl.ds(s * RPT, RPT)])
    pltpu.sync_copy(dst_hbm.at[w], dst_v)
    for i in range(B // 16):
        ones_v[pl.ds(i * 16, 16)] = jnp.ones((16,), jnp.float32)
    plsc.subcore_barrier()

    def body(j, carry):
        pltpu.sync_copy(ones_v, acc_sp.at[dst_v.at[j]], add=True)
        return carry

    lax.fori_loop(0, NB, body, 0)
    plsc.subcore_barrier()
    pltpu.sync_copy(acc_sp.at[pl.ds(s * RPT, RPT)],
                    out_hbm.at[c, pl.ds(s * RPT, RPT)])


def _agg_pass(y_hbm, out_hbm, p, s, c, w, src_hbm, dst_hbm, src_v, dst_v,
              rows, gsem, ssem, y_sp, acc_sp, zeros_hbm):
    nbh = NB // 2
    pltpu.async_copy(y_hbm.at[pl.ds(s * SRT, SRT), pl.ds(F * p, F)],
                     y_sp.at[pl.ds(s * SRT, SRT)], gsem[0])
    pltpu.async_copy(zeros_hbm, acc_sp.at[pl.ds(s * RPT, RPT)], gsem[1])
    pltpu.async_copy(src_hbm.at[w, pl.ds(0, nbh)], src_v, gsem[2])
    pltpu.async_copy(dst_hbm.at[w, pl.ds(0, nbh)], dst_v, gsem[3])
    pltpu.make_async_copy(y_hbm.at[pl.ds(s * SRT, SRT), pl.ds(F * p, F)],
                          y_sp.at[pl.ds(s * SRT, SRT)], gsem[0]).wait()
    pltpu.make_async_copy(zeros_hbm, acc_sp.at[pl.ds(s * RPT, RPT)],
                          gsem[1]).wait()
    pltpu.make_async_copy(src_hbm.at[w, pl.ds(0, nbh)], src_v,
                          gsem[2]).wait()
    pltpu.make_async_copy(dst_hbm.at[w, pl.ds(0, nbh)], dst_v,
                          gsem[3]).wait()
    plsc.subcore_barrier()

    def g(jb, k):
        pltpu.async_copy(y_sp.at[src_v.at[jb]], rows[k], gsem[k])

    def gwait(k):
        pltpu.make_async_copy(y_sp.at[src_v.at[0]], rows[k], gsem[k]).wait()

    def sc(jb, k):
        pltpu.async_copy(rows[k], acc_sp.at[dst_v.at[jb]], ssem[k],
                         add=True)

    def scwait(k):
        pltpu.make_async_copy(rows[k], acc_sp.at[dst_v.at[0]],
                              ssem[k]).wait()

    for h in range(2):
        if h:
            pltpu.sync_copy(src_hbm.at[w, pl.ds(h * nbh, nbh)], src_v)
            pltpu.sync_copy(dst_hbm.at[w, pl.ds(h * nbh, nbh)], dst_v)
        g(0, 0)
        g(1, 1)
        gwait(0); sc(0, 0); g(2, 2)
        gwait(1); sc(1, 1); g(3, 3)
        gwait(2); sc(2, 2); scwait(0); g(4, 0)
        gwait(3); sc(3, 3); scwait(1); g(5, 1)

        def body(i, carry):
            j0 = 4 * i
            gwait(0); sc(j0, 0); scwait(2); g(jnp.minimum(j0 + 2, nbh - 1), 2)
            gwait(1); sc(j0 + 1, 1); scwait(3)
            g(jnp.minimum(j0 + 3, nbh - 1), 3)
            gwait(2); sc(j0 + 2, 2); scwait(0)
            g(jnp.minimum(j0 + 4, nbh - 1), 0)
            gwait(3); sc(j0 + 3, 3); scwait(1)
            g(jnp.minimum(j0 + 5, nbh - 1), 1)
            return carry

        lax.fori_loop(1, nbh // 4, body, 0)
        gwait(0)
        gwait(1)
        scwait(2)
        scwait(3)

    plsc.subcore_barrier()
    pltpu.sync_copy(acc_sp.at[pl.ds(s * RPT, RPT)],
                    out_hbm.at[c, pl.ds(s * RPT, RPT), pl.ds(F * p, F)])


def _make_agg(nparts):
    scratch = [pltpu.VMEM((NB // 2, B), jnp.int32),
               pltpu.VMEM((NB // 2, B), jnp.int32),
               pltpu.VMEM((B, F), jnp.float32),
               pltpu.VMEM((B, F), jnp.float32),
               pltpu.VMEM((B, F), jnp.float32),
               pltpu.VMEM((B, F), jnp.float32),
               pltpu.VMEM_SHARED((N, F), jnp.float32),
               pltpu.VMEM_SHARED((NPAD, F), jnp.float32),
               pltpu.SemaphoreType.DMA,
               pltpu.SemaphoreType.DMA,
               pltpu.SemaphoreType.DMA,
               pltpu.SemaphoreType.DMA,
               pltpu.SemaphoreType.DMA,
               pltpu.SemaphoreType.DMA,
               pltpu.SemaphoreType.DMA,
               pltpu.SemaphoreType.DMA]
    out_type = jax.ShapeDtypeStruct((2, NPAD, HID), jnp.float32)

    @functools.partial(pl.kernel, mesh=_MESH, compiler_params=_SC_PARAMS,
                       out_type=out_type, scratch_types=scratch)
    def agg(y_hbm, src_hbm, dst_hbm, zeros_hbm, out_hbm,
            src_v, dst_v, r0, r1, r2, r3, y_sp, acc_sp,
            g0, g1, g2, g3, s0, s1, s2, s3):
        c = lax.axis_index("c")
        s = lax.axis_index("s")
        w = c * 16 + s
        rows = [r0, r1, r2, r3]
        gsem = [g0, g1, g2, g3]
        ssem = [s0, s1, s2, s3]
        for p in range(nparts):
            _agg_pass(y_hbm, out_hbm, p, s, c, w, src_hbm, dst_hbm,
                      src_v, dst_v, rows, gsem, ssem, y_sp, acc_sp,
                      zeros_hbm)

    return agg


_agg2 = _make_agg(2)
_agg1 = _make_agg(1)


def kernel(x, edge_index, W0, b0, W1, b1):
    src = edge_index[0]
    dst = edge_index[1]
    pad = EPAD - E
    srcp = jnp.concatenate(
        [src, jnp.zeros((pad,), jnp.int32)]).reshape(NW, NB, B)
    dstp = jnp.concatenate(
        [dst, jnp.full((pad,), DUMMY, jnp.int32)]).reshape(NW, NB, B)

    z_deg = jnp.zeros((RPT,), jnp.float32)
    deg = _deg_kernel(dstp, z_deg)
    xw0 = _mm(x, W0)
    y0, dinv = _scale0(deg.reshape(2, NPAD, 1), xw0)

    z_acc = jnp.zeros((RPT, F), jnp.float32)
    acc0 = _agg2(y0, srcp, dstp, z_acc)
    y1 = _mid(acc0, y0, dinv, b0.reshape(1, HID), W1)
    acc1 = _agg1(y1, srcp, dstp, z_acc)
    out = _fin(acc1, y1, dinv, b1.reshape(1, N_CLS))
    return out


# speedup vs baseline: 33.7192x; 1.0131x over previous
import functools

import jax
import jax.numpy as jnp
from jax import lax
from jax.experimental import pallas as pl
from jax.experimental.pallas import tpu as pltpu
from jax.experimental.pallas import tpu_sc as plsc

N = 10000
E = 320000
D_IN = 128
HID = 128
N_CLS = 64
F = 64

NPAD = 10240
NW = 32
NB = 80
B = 128
EPAD = NW * NB * B
RPT = NPAD // 16
SRT = N // 16
DUMMY = N + 16

MT = 1000


def _mm_body(x_ref, w_ref, o_ref):
    o_ref[...] = jnp.dot(x_ref[...], w_ref[...],
                         preferred_element_type=jnp.float32)


def _mm(x, w):
    m, k = x.shape
    n = w.shape[1]
    return pl.pallas_call(
        _mm_body,
        grid=(m // MT,),
        in_specs=[pl.BlockSpec((MT, k), lambda i: (i, 0)),
                  pl.BlockSpec((k, n), lambda i: (0, 0))],
        out_specs=pl.BlockSpec((MT, n), lambda i: (i, 0)),
        out_shape=jax.ShapeDtypeStruct((m, n), jnp.float32),
    )(x, w)


def _scale0_body(deg_ref, xw_ref, y_ref, dinv_ref):
    d = deg_ref[0, :, :1] + deg_ref[1, :, :1] + 1.0
    dinv = lax.rsqrt(d)
    dinv_ref[...] = dinv
    y_ref[...] = xw_ref[...] * dinv


def _scale0(deg, xw):
    return pl.pallas_call(
        _scale0_body,
        grid=(N // MT,),
        in_specs=[pl.BlockSpec((2, MT, HID), lambda i: (0, i, 0)),
                  pl.BlockSpec((MT, HID), lambda i: (i, 0))],
        out_specs=[pl.BlockSpec((MT, HID), lambda i: (i, 0)),
                   pl.BlockSpec((MT, 1), lambda i: (i, 0))],
        out_shape=[jax.ShapeDtypeStruct((N, HID), jnp.float32),
                   jax.ShapeDtypeStruct((N, 1), jnp.float32)],
    )(deg, xw)


def _mid_body(acc_ref, y0_ref, dinv_ref, b0_ref, w1_ref, y1_ref):
    dinv = dinv_ref[...]
    h = acc_ref[0] + acc_ref[1] + y0_ref[...]
    h = jnp.maximum(dinv * h + b0_ref[...], 0.0)
    y1 = jnp.dot(h, w1_ref[...],
                 preferred_element_type=jnp.float32) * dinv
    y1_ref[...] = jnp.concatenate(
        [y1, jnp.zeros((MT, HID - N_CLS), jnp.float32)], axis=1)


def _mid(acc, y0, dinv, b0, w1):
    return pl.pallas_call(
        _mid_body,
        grid=(N // MT,),
        in_specs=[pl.BlockSpec((2, MT, HID), lambda i: (0, i, 0)),
                  pl.BlockSpec((MT, HID), lambda i: (i, 0)),
                  pl.BlockSpec((MT, 1), lambda i: (i, 0)),
                  pl.BlockSpec((1, HID), lambda i: (0, 0)),
                  pl.BlockSpec((HID, N_CLS), lambda i: (0, 0))],
        out_specs=pl.BlockSpec((MT, HID), lambda i: (i, 0)),
        out_shape=jax.ShapeDtypeStruct((N, HID), jnp.float32),
    )(acc, y0, dinv, b0, w1)


def _fin_body(acc_ref, y1_ref, dinv_ref, b1_ref, o_ref):
    agg = acc_ref[0, :, :N_CLS] + acc_ref[1, :, :N_CLS]
    o_ref[...] = (dinv_ref[...] * (agg + y1_ref[:, :N_CLS]) + b1_ref[...])


def _fin(acc, y1, dinv, b1):
    return pl.pallas_call(
        _fin_body,
        grid=(N // MT,),
        in_specs=[pl.BlockSpec((2, MT, HID), lambda i: (0, i, 0)),
                  pl.BlockSpec((MT, HID), lambda i: (i, 0)),
                  pl.BlockSpec((MT, 1), lambda i: (i, 0)),
                  pl.BlockSpec((1, N_CLS), lambda i: (0, 0))],
        out_specs=pl.BlockSpec((MT, N_CLS), lambda i: (i, 0)),
        out_shape=jax.ShapeDtypeStruct((N, N_CLS), jnp.float32),
    )(acc, y1, dinv, b1)


_MESH = plsc.VectorSubcoreMesh(core_axis_name="c", subcore_axis_name="s")
_SC_PARAMS = pltpu.CompilerParams(use_tc_tiling_on_sc=False)


@functools.partial(
    pl.kernel, mesh=_MESH,
    compiler_params=_SC_PARAMS,
    out_type=jax.ShapeDtypeStruct((2, NPAD, HID), jnp.float32),
    scratch_types=[pltpu.VMEM((NB, B), jnp.int32),
                   pltpu.VMEM((B, 16), jnp.float32),
                   pltpu.VMEM_SHARED((NPAD, 16), jnp.float32)])
def _deg_kernel(dst_hbm, ones_hbm, zeros_hbm, out_hbm, dst_v, ones_v,
                acc_sp):
    c = lax.axis_index("c")
    s = lax.axis_index("s")
    w = c * 16 + s
    pltpu.sync_copy(zeros_hbm, acc_sp.at[pl.ds(s * RPT, RPT)])
    pltpu.sync_copy(dst_hbm.at[w], dst_v)
    pltpu.sync_copy(ones_hbm, ones_v)
    plsc.subcore_barrier()

    def body(j, carry):
        pltpu.sync_copy(ones_v, acc_sp.at[dst_v.at[j]], add=True)
        return carry

    lax.fori_loop(0, NB, body, 0)
    plsc.subcore_barrier()
    pltpu.sync_copy(acc_sp.at[pl.ds(s * RPT, RPT)],
                    out_hbm.at[c, pl.ds(s * RPT, RPT), pl.ds(0, 16)])


def _agg_pass(y_hbm, out_hbm, p, s, c, w, src_hbm, dst_hbm, src_v, dst_v,
              rows, gsem, ssem, y_sp, acc_sp, zeros_hbm):
    nbh = NB // 2
    pltpu.async_copy(y_hbm.at[pl.ds(s * SRT, SRT), pl.ds(F * p, F)],
                     y_sp.at[pl.ds(s * SRT, SRT)], gsem[0])
    pltpu.async_copy(zeros_hbm, acc_sp.at[pl.ds(s * RPT, RPT)], gsem[1])
    pltpu.async_copy(src_hbm.at[w, pl.ds(0, nbh)], src_v, gsem[2])
    pltpu.async_copy(dst_hbm.at[w, pl.ds(0, nbh)], dst_v, gsem[3])
    pltpu.make_async_copy(y_hbm.at[pl.ds(s * SRT, SRT), pl.ds(F * p, F)],
                          y_sp.at[pl.ds(s * SRT, SRT)], gsem[0]).wait()
    pltpu.make_async_copy(zeros_hbm, acc_sp.at[pl.ds(s * RPT, RPT)],
                          gsem[1]).wait()
    pltpu.make_async_copy(src_hbm.at[w, pl.ds(0, nbh)], src_v,
                          gsem[2]).wait()
    pltpu.make_async_copy(dst_hbm.at[w, pl.ds(0, nbh)], dst_v,
                          gsem[3]).wait()
    plsc.subcore_barrier()

    def g(jb, k):
        pltpu.async_copy(y_sp.at[src_v.at[jb]], rows[k], gsem[k])

    def gwait(k):
        pltpu.make_async_copy(y_sp.at[src_v.at[0]], rows[k], gsem[k]).wait()

    def sc(jb, k):
        pltpu.async_copy(rows[k], acc_sp.at[dst_v.at[jb]], ssem[k],
                         add=True)

    def scwait(k):
        pltpu.make_async_copy(rows[k], acc_sp.at[dst_v.at[0]],
                              ssem[k]).wait()

    for h in range(2):
        if h:
            pltpu.sync_copy(src_hbm.at[w, pl.ds(h * nbh, nbh)], src_v)
            pltpu.sync_copy(dst_hbm.at[w, pl.ds(h * nbh, nbh)], dst_v)
        g(0, 0)
        g(1, 1)
        gwait(0); sc(0, 0); g(2, 2)
        gwait(1); sc(1, 1); g(3, 3)
        gwait(2); sc(2, 2); scwait(0); g(4, 0)
        gwait(3); sc(3, 3); scwait(1); g(5, 1)

        def body(i, carry):
            j0 = 4 * i
            gwait(0); sc(j0, 0); scwait(2); g(jnp.minimum(j0 + 2, nbh - 1), 2)
            gwait(1); sc(j0 + 1, 1); scwait(3)
            g(jnp.minimum(j0 + 3, nbh - 1), 3)
            gwait(2); sc(j0 + 2, 2); scwait(0)
            g(jnp.minimum(j0 + 4, nbh - 1), 0)
            gwait(3); sc(j0 + 3, 3); scwait(1)
            g(jnp.minimum(j0 + 5, nbh - 1), 1)
            return carry

        lax.fori_loop(1, nbh // 4, body, 0)
        gwait(0)
        gwait(1)
        scwait(2)
        scwait(3)

    plsc.subcore_barrier()
    pltpu.sync_copy(acc_sp.at[pl.ds(s * RPT, RPT)],
                    out_hbm.at[c, pl.ds(s * RPT, RPT), pl.ds(F * p, F)])


def _make_agg(nparts):
    scratch = [pltpu.VMEM((NB // 2, B), jnp.int32),
               pltpu.VMEM((NB // 2, B), jnp.int32),
               pltpu.VMEM((B, F), jnp.float32),
               pltpu.VMEM((B, F), jnp.float32),
               pltpu.VMEM((B, F), jnp.float32),
               pltpu.VMEM((B, F), jnp.float32),
               pltpu.VMEM_SHARED((N, F), jnp.float32),
               pltpu.VMEM_SHARED((NPAD, F), jnp.float32),
               pltpu.SemaphoreType.DMA,
               pltpu.SemaphoreType.DMA,
               pltpu.SemaphoreType.DMA,
               pltpu.SemaphoreType.DMA,
               pltpu.SemaphoreType.DMA,
               pltpu.SemaphoreType.DMA,
               pltpu.SemaphoreType.DMA,
               pltpu.SemaphoreType.DMA]
    out_type = jax.ShapeDtypeStruct((2, NPAD, HID), jnp.float32)

    @functools.partial(pl.kernel, mesh=_MESH, compiler_params=_SC_PARAMS,
                       out_type=out_type, scratch_types=scratch)
    def agg(y_hbm, src_hbm, dst_hbm, zeros_hbm, out_hbm,
            src_v, dst_v, r0, r1, r2, r3, y_sp, acc_sp,
            g0, g1, g2, g3, s0, s1, s2, s3):
        c = lax.axis_index("c")
        s = lax.axis_index("s")
        w = c * 16 + s
        rows = [r0, r1, r2, r3]
        gsem = [g0, g1, g2, g3]
        ssem = [s0, s1, s2, s3]
        for p in range(nparts):
            _agg_pass(y_hbm, out_hbm, p, s, c, w, src_hbm, dst_hbm,
                      src_v, dst_v, rows, gsem, ssem, y_sp, acc_sp,
                      zeros_hbm)

    return agg


_agg2 = _make_agg(2)
_agg1 = _make_agg(1)


def kernel(x, edge_index, W0, b0, W1, b1):
    src = edge_index[0]
    dst = edge_index[1]
    pad = EPAD - E
    srcp = jnp.concatenate(
        [src, jnp.zeros((pad,), jnp.int32)]).reshape(NW, NB, B)
    dstp = jnp.concatenate(
        [dst, jnp.full((pad,), DUMMY, jnp.int32)]).reshape(NW, NB, B)

    z_deg = jnp.zeros((RPT, 16), jnp.float32)
    ones = jnp.ones((B, 16), jnp.float32)
    deg = _deg_kernel(dstp, ones, z_deg)
    xw0 = _mm(x, W0)
    y0, dinv = _scale0(deg, xw0)

    z_acc = jnp.zeros((RPT, F), jnp.float32)
    acc0 = _agg2(y0, srcp, dstp, z_acc)
    y1 = _mid(acc0, y0, dinv, b0.reshape(1, HID), W1)
    acc1 = _agg1(y1, srcp, dstp, z_acc)
    out = _fin(acc1, y1, dinv, b1.reshape(1, N_CLS))
    return out
